# Initial kernel scaffold; baseline (speedup 1.0000x reference)
import jax
import jax.numpy as jnp
from jax.experimental import pallas as pl


def kernel(x, W):
    raise NotImplementedError("write your pallas kernel here")


# speedup vs baseline: 2.7894x; 2.7894x over previous
import jax
import jax.numpy as jnp
from jax.experimental import pallas as pl
from jax.experimental.pallas import tpu as pltpu

TB = 1024


def _router_body(x_ref, w_ref, idx_ref, val_ref, aux_ref, imp_acc, load_acc):
    step = pl.program_id(0)
    nsteps = pl.num_programs(0)

    x = x_ref[...]
    w = w_ref[...]
    logits = jnp.dot(x, w, preferred_element_type=jnp.float32)

    m = jnp.max(logits, axis=-1, keepdims=True)
    i1 = jnp.argmax(logits, axis=-1).astype(jnp.int32)
    e = jnp.exp(logits - m)
    s = jnp.sum(e, axis=-1, keepdims=True)
    inv_s = 1.0 / s
    probs = e * inv_s

    cols = jax.lax.broadcasted_iota(jnp.int32, logits.shape, 1)
    hit1 = cols == i1[:, None]
    masked = jnp.where(hit1, -jnp.inf, logits)
    m2 = jnp.max(masked, axis=-1, keepdims=True)
    i2 = jnp.argmax(masked, axis=-1).astype(jnp.int32)
    v1 = inv_s[:, 0]
    v2 = (jnp.exp(m2 - m) * inv_s)[:, 0]

    idx_ref[0:1, :] = i1[None, :]
    idx_ref[1:2, :] = i2[None, :]
    val_ref[0:1, :] = v1[None, :]
    val_ref[1:2, :] = v2[None, :]

    blk_imp = jnp.sum(probs, axis=0, keepdims=True)
    blk_load = jnp.sum(hit1.astype(jnp.float32), axis=0, keepdims=True)

    @pl.when(step == 0)
    def _init():
        imp_acc[...] = blk_imp
        load_acc[...] = blk_load

    @pl.when(step != 0)
    def _accum():
        imp_acc[...] += blk_imp
        load_acc[...] += blk_load

    @pl.when(step == nsteps - 1)
    def _finalize():
        S = nsteps * TB
        E = w_ref.shape[1]
        scale = E / (float(S) * float(S))
        aux_ref[0, 0] = scale * jnp.sum(imp_acc[...] * load_acc[...])


def kernel(x, W):
    S, D = x.shape
    E = W.shape[1]
    grid = (S // TB,)

    idx_t, val_t, aux = pl.pallas_call(
        _router_body,
        grid=grid,
        in_specs=[
            pl.BlockSpec((TB, D), lambda i: (i, 0)),
            pl.BlockSpec((D, E), lambda i: (0, 0)),
        ],
        out_specs=[
            pl.BlockSpec((2, TB), lambda i: (0, i)),
            pl.BlockSpec((2, TB), lambda i: (0, i)),
            pl.BlockSpec(memory_space=pltpu.SMEM),
        ],
        out_shape=[
            jax.ShapeDtypeStruct((2, S), jnp.int32),
            jax.ShapeDtypeStruct((2, S), jnp.float32),
            jax.ShapeDtypeStruct((1, 1), jnp.float32),
        ],
        scratch_shapes=[
            pltpu.VMEM((1, E), jnp.float32),
            pltpu.VMEM((1, E), jnp.float32),
        ],
        compiler_params=pltpu.CompilerParams(
            dimension_semantics=("arbitrary",),
        ),
    )(x, W)

    return idx_t.T, val_t.T, aux[0, 0]


# speedup vs baseline: 4.7393x; 1.6990x over previous
import jax
import jax.numpy as jnp
from jax.experimental import pallas as pl
from jax.experimental.pallas import tpu as pltpu

TB = 1024


def _router_body(x_ref, w_ref, idx_ref, val_ref, aux_ref, imp_acc, load_acc):
    step = pl.program_id(0)
    nsteps = pl.num_programs(0)

    x = x_ref[...]
    w = w_ref[...]
    logits_tm = jnp.dot(x, w, preferred_element_type=jnp.float32)
    l = logits_tm.T

    m = jnp.max(l, axis=0, keepdims=True)
    i1 = jnp.argmax(l, axis=0).astype(jnp.int32)
    e = jnp.exp(l - m)
    s = jnp.sum(e, axis=0, keepdims=True)
    inv_s = 1.0 / s
    probs = e * inv_s

    rows = jax.lax.broadcasted_iota(jnp.int32, l.shape, 0)
    hit1 = rows == i1[None, :]
    masked = jnp.where(hit1, -jnp.inf, l)
    m2 = jnp.max(masked, axis=0, keepdims=True)
    i2 = jnp.argmax(masked, axis=0).astype(jnp.int32)
    v1 = inv_s
    v2 = jnp.exp(m2 - m) * inv_s

    idx_ref[0:1, :] = i1[None, :]
    idx_ref[1:2, :] = i2[None, :]
    val_ref[0:1, :] = v1
    val_ref[1:2, :] = v2

    blk_imp = jnp.sum(probs, axis=1, keepdims=True)
    blk_load = jnp.sum(hit1.astype(jnp.float32), axis=1, keepdims=True)

    @pl.when(step == 0)
    def _init():
        imp_acc[...] = blk_imp
        load_acc[...] = blk_load

    @pl.when(step != 0)
    def _accum():
        imp_acc[...] += blk_imp
        load_acc[...] += blk_load

    @pl.when(step == nsteps - 1)
    def _finalize():
        S = nsteps * TB
        E = w_ref.shape[1]
        scale = E / (float(S) * float(S))
        aux_ref[0, 0] = scale * jnp.sum(imp_acc[...] * load_acc[...])


def kernel(x, W):
    S, D = x.shape
    E = W.shape[1]
    grid = (S // TB,)

    idx_t, val_t, aux = pl.pallas_call(
        _router_body,
        grid=grid,
        in_specs=[
            pl.BlockSpec((TB, D), lambda i: (i, 0)),
            pl.BlockSpec((D, E), lambda i: (0, 0)),
        ],
        out_specs=[
            pl.BlockSpec((2, TB), lambda i: (0, i)),
            pl.BlockSpec((2, TB), lambda i: (0, i)),
            pl.BlockSpec(memory_space=pltpu.SMEM),
        ],
        out_shape=[
            jax.ShapeDtypeStruct((2, S), jnp.int32),
            jax.ShapeDtypeStruct((2, S), jnp.float32),
            jax.ShapeDtypeStruct((1, 1), jnp.float32),
        ],
        scratch_shapes=[
            pltpu.VMEM((E, 1), jnp.float32),
            pltpu.VMEM((E, 1), jnp.float32),
        ],
        compiler_params=pltpu.CompilerParams(
            dimension_semantics=("arbitrary",),
        ),
    )(x, W)

    return idx_t.T, val_t.T, aux[0, 0]


# speedup vs baseline: 5.9109x; 1.2472x over previous
import jax
import jax.numpy as jnp
from jax.experimental import pallas as pl
from jax.experimental.pallas import tpu as pltpu

TB = 2048


def _router_body(x_ref, w_ref, idx_ref, val_ref, aux_ref, imp_acc, load_acc):
    step = pl.program_id(0)
    nsteps = pl.num_programs(0)

    x = x_ref[...]
    w = w_ref[...]
    logits_tm = jnp.dot(x, w, preferred_element_type=jnp.float32)
    l = logits_tm.T

    m = jnp.max(l, axis=0, keepdims=True)
    i1 = jnp.argmax(l, axis=0).astype(jnp.int32)
    e = jnp.exp(l - m)
    s = jnp.sum(e, axis=0, keepdims=True)
    inv_s = 1.0 / s
    probs = e * inv_s

    rows = jax.lax.broadcasted_iota(jnp.int32, l.shape, 0)
    hit1 = rows == i1[None, :]
    masked = jnp.where(hit1, -jnp.inf, l)
    m2 = jnp.max(masked, axis=0, keepdims=True)
    i2 = jnp.argmax(masked, axis=0).astype(jnp.int32)
    v1 = inv_s
    v2 = jnp.exp(m2 - m) * inv_s

    idx_ref[0:1, :] = i1[None, :]
    idx_ref[1:2, :] = i2[None, :]
    val_ref[0:1, :] = v1
    val_ref[1:2, :] = v2

    blk_imp = jnp.sum(probs, axis=1, keepdims=True)
    blk_load = jnp.sum(hit1.astype(jnp.float32), axis=1, keepdims=True)

    @pl.when(step == 0)
    def _init():
        imp_acc[...] = blk_imp
        load_acc[...] = blk_load

    @pl.when(step != 0)
    def _accum():
        imp_acc[...] += blk_imp
        load_acc[...] += blk_load

    @pl.when(step == nsteps - 1)
    def _finalize():
        S = nsteps * TB
        E = w_ref.shape[1]
        scale = E / (float(S) * float(S))
        aux_ref[0, 0] = scale * jnp.sum(imp_acc[...] * load_acc[...])


def kernel(x, W):
    S, D = x.shape
    E = W.shape[1]
    grid = (S // TB,)

    idx_t, val_t, aux = pl.pallas_call(
        _router_body,
        grid=grid,
        in_specs=[
            pl.BlockSpec((TB, D), lambda i: (i, 0)),
            pl.BlockSpec((D, E), lambda i: (0, 0)),
        ],
        out_specs=[
            pl.BlockSpec((2, TB), lambda i: (0, i)),
            pl.BlockSpec((2, TB), lambda i: (0, i)),
            pl.BlockSpec(memory_space=pltpu.SMEM),
        ],
        out_shape=[
            jax.ShapeDtypeStruct((2, S), jnp.int32),
            jax.ShapeDtypeStruct((2, S), jnp.float32),
            jax.ShapeDtypeStruct((1, 1), jnp.float32),
        ],
        scratch_shapes=[
            pltpu.VMEM((E, 1), jnp.float32),
            pltpu.VMEM((E, 1), jnp.float32),
        ],
        compiler_params=pltpu.CompilerParams(
            dimension_semantics=("arbitrary",),
        ),
    )(x, W)

    return idx_t.T, val_t.T, aux[0, 0]


# speedup vs baseline: 6.3495x; 1.0742x over previous
import jax
import jax.numpy as jnp
from jax.experimental import pallas as pl
from jax.experimental.pallas import tpu as pltpu

TB = 4096


def _router_body(x_ref, w_ref, idx_ref, val_ref, aux_ref, imp_acc, load_acc):
    step = pl.program_id(0)
    nsteps = pl.num_programs(0)

    x = x_ref[...]
    w = w_ref[...]
    logits_tm = jnp.dot(x, w, preferred_element_type=jnp.float32)
    l = logits_tm.T

    m = jnp.max(l, axis=0, keepdims=True)
    i1 = jnp.argmax(l, axis=0).astype(jnp.int32)
    e = jnp.exp(l - m)
    s = jnp.sum(e, axis=0, keepdims=True)
    inv_s = 1.0 / s
    probs = e * inv_s

    rows = jax.lax.broadcasted_iota(jnp.int32, l.shape, 0)
    hit1 = rows == i1[None, :]
    masked = jnp.where(hit1, -jnp.inf, l)
    m2 = jnp.max(masked, axis=0, keepdims=True)
    i2 = jnp.argmax(masked, axis=0).astype(jnp.int32)
    v1 = inv_s
    v2 = jnp.exp(m2 - m) * inv_s

    idx_ref[0:1, :] = i1[None, :]
    idx_ref[1:2, :] = i2[None, :]
    val_ref[0:1, :] = v1
    val_ref[1:2, :] = v2

    blk_imp = jnp.sum(probs, axis=1, keepdims=True)
    blk_load = jnp.sum(hit1.astype(jnp.float32), axis=1, keepdims=True)

    @pl.when(step == 0)
    def _init():
        imp_acc[...] = blk_imp
        load_acc[...] = blk_load

    @pl.when(step != 0)
    def _accum():
        imp_acc[...] += blk_imp
        load_acc[...] += blk_load

    @pl.when(step == nsteps - 1)
    def _finalize():
        S = nsteps * TB
        E = w_ref.shape[1]
        scale = E / (float(S) * float(S))
        aux_ref[0, 0] = scale * jnp.sum(imp_acc[...] * load_acc[...])


def kernel(x, W):
    S, D = x.shape
    E = W.shape[1]
    grid = (S // TB,)

    idx_t, val_t, aux = pl.pallas_call(
        _router_body,
        grid=grid,
        in_specs=[
            pl.BlockSpec((TB, D), lambda i: (i, 0)),
            pl.BlockSpec((D, E), lambda i: (0, 0)),
        ],
        out_specs=[
            pl.BlockSpec((2, TB), lambda i: (0, i)),
            pl.BlockSpec((2, TB), lambda i: (0, i)),
            pl.BlockSpec(memory_space=pltpu.SMEM),
        ],
        out_shape=[
            jax.ShapeDtypeStruct((2, S), jnp.int32),
            jax.ShapeDtypeStruct((2, S), jnp.float32),
            jax.ShapeDtypeStruct((1, 1), jnp.float32),
        ],
        scratch_shapes=[
            pltpu.VMEM((E, 1), jnp.float32),
            pltpu.VMEM((E, 1), jnp.float32),
        ],
        compiler_params=pltpu.CompilerParams(
            dimension_semantics=("arbitrary",),
        ),
    )(x, W)

    return idx_t.T, val_t.T, aux[0, 0]
